# baseline (speedup 1.0000x reference)
import functools

import jax
import jax.numpy as jnp
from jax import lax
from jax.experimental import pallas as pl
from jax.experimental.pallas import tpu as pltpu
from jax.experimental.pallas import tpu_sc as plsc

N = 10000
F = 128
H = 256
HF = H // 2
E = 160000
EPS = 1e-5

NS = 16
EPT = E // NS
CW = 104
NCHUNK = 98
EPAD = NCHUNK * CW - EPT
NPAD = 10008
RPT = 632
RPT_LAST = NPAD - 15 * RPT

BN = 1000
NB = N // BN


def _sc_agg_body(h_hbm, src_hbm, dst_hbm, zeros_hbm, out_hbm,
                 src_v, dst_v, rows_v0, rows_v1, acc, sem0, sem1):
    c = lax.axis_index("c")
    s = lax.axis_index("s")
    pltpu.sync_copy(src_hbm.at[s], src_v)
    pltpu.sync_copy(dst_hbm.at[s], dst_v)
    pl.when(s < 15)(lambda: pltpu.sync_copy(
        zeros_hbm, acc.at[pl.ds(s * RPT, RPT)]))
    pl.when(s == 15)(lambda: pltpu.sync_copy(
        zeros_hbm.at[pl.ds(0, RPT_LAST)], acc.at[pl.ds(15 * RPT, RPT_LAST)]))
    plsc.subcore_barrier()

    def half(h2, o2):
        rows = (rows_v0, rows_v1)
        sems = (sem0, sem1)

        def sidx(j):
            return src_v.at[pl.ds(pl.multiple_of(j * CW, 8), CW)]

        def gather(j, b):
            pltpu.async_copy(h2.at[sidx(j)], rows[b], sems[b])

        def drain_scatter(j, b):
            pltpu.make_async_copy(h2.at[sidx(j)], rows[b], sems[b]).wait()
            pltpu.sync_copy(rows[b], acc.at[dst_v.at[j]], add=True)

        gather(0, 0)

        def body2(t, carry):
            j = 2 * t
            gather(j + 1, 1)
            drain_scatter(j, 0)
            gather(lax.rem(j + 2, NCHUNK), 0)
            drain_scatter(j + 1, 1)
            return carry
        lax.fori_loop(0, NCHUNK // 2, body2, 0)
        pltpu.make_async_copy(h2.at[sidx(0)], rows[0], sems[0]).wait()
        plsc.subcore_barrier()
        pl.when(s < 15)(lambda: pltpu.sync_copy(
            acc.at[pl.ds(s * RPT, RPT)], o2.at[pl.ds(s * RPT, RPT)]))
        pl.when(s == 15)(lambda: pltpu.sync_copy(
            acc.at[pl.ds(15 * RPT, RPT_LAST)],
            o2.at[pl.ds(15 * RPT, RPT_LAST)]))

    pl.when(c == 0)(lambda: half(h_hbm.at[0], out_hbm.at[0]))
    pl.when(c == 1)(lambda: half(h_hbm.at[1], out_hbm.at[1]))


def _make_sc_agg():
    mesh = plsc.VectorSubcoreMesh(core_axis_name="c", subcore_axis_name="s")
    return pl.kernel(
        _sc_agg_body,
        out_type=jax.ShapeDtypeStruct((2, NPAD, HF), jnp.float32),
        mesh=mesh,
        scratch_types=[
            pltpu.VMEM((NCHUNK * CW,), jnp.int32),
            pltpu.VMEM((NCHUNK, CW), jnp.int32),
            pltpu.VMEM((CW, HF), jnp.float32),
            pltpu.VMEM((CW, HF), jnp.float32),
            pltpu.VMEM_SHARED((NPAD, HF), jnp.float32),
            pltpu.SemaphoreType.DMA,
            pltpu.SemaphoreType.DMA,
        ],
    )


def _transform_body(x_ref, wt_ref, bt_ref, y_ref, st_ref):
    i = pl.program_id(0)
    y = jnp.dot(x_ref[...], wt_ref[...], preferred_element_type=jnp.float32)
    y = y + bt_ref[...]
    y_ref[...] = y
    st = jnp.concatenate(
        [jnp.sum(y, axis=0, keepdims=True),
         jnp.sum(y * y, axis=0, keepdims=True)], axis=0)

    @pl.when(i == 0)
    def _():
        st_ref[...] = st

    @pl.when(i > 0)
    def _():
        st_ref[...] += st


def _layer_body(h_ref, a_ref, w1_ref, w2_ref, y_ref, st_ref):
    i = pl.program_id(0)
    s0 = h_ref[0] + a_ref[0]
    s1 = h_ref[1] + a_ref[1]
    u = (jnp.dot(s0, w1_ref[:HF, :], preferred_element_type=jnp.float32)
         + jnp.dot(s1, w1_ref[HF:, :], preferred_element_type=jnp.float32))
    u = jnp.maximum(u, 0.0)
    y = jnp.dot(u, w2_ref[...], preferred_element_type=jnp.float32)
    y = jnp.maximum(y, 0.0)
    y_ref[...] = y
    st = jnp.concatenate(
        [jnp.sum(y, axis=0, keepdims=True),
         jnp.sum(y * y, axis=0, keepdims=True)], axis=0)

    @pl.when(i == 0)
    def _():
        st_ref[...] = st

    @pl.when(i > 0)
    def _():
        st_ref[...] += st


def _norm(y, st, g, b):
    m = st[0:1, :] * (1.0 / N)
    v = st[1:2, :] * (1.0 / N) - m * m
    inv = lax.rsqrt(v + EPS)
    return (y - m) * (inv * g) + b


def _norm_split_body(y_ref, st_ref, g_ref, b_ref, o_ref):
    hn = _norm(y_ref[...], st_ref[...], g_ref[...], b_ref[...])
    o_ref[0] = hn[:, :HF]
    o_ref[1] = hn[:, HF:]


def _norm_full_body(y_ref, st_ref, g_ref, b_ref, o_ref):
    o_ref[...] = _norm(y_ref[...], st_ref[...], g_ref[...], b_ref[...])


_vec_spec = pl.BlockSpec((1, H), lambda i: (0, 0))
_st_spec = pl.BlockSpec((2, H), lambda i: (0, 0))
_y_spec = pl.BlockSpec((BN, H), lambda i: (i, 0))
_split_spec = pl.BlockSpec((2, BN, HF), lambda i: (0, i, 0))

_transform = pl.pallas_call(
    _transform_body,
    grid=(NB,),
    in_specs=[pl.BlockSpec((BN, F), lambda i: (i, 0)),
              pl.BlockSpec((F, H), lambda i: (0, 0)),
              _vec_spec],
    out_specs=[_y_spec, _st_spec],
    out_shape=[jax.ShapeDtypeStruct((N, H), jnp.float32),
               jax.ShapeDtypeStruct((2, H), jnp.float32)],
)

_layer = pl.pallas_call(
    _layer_body,
    grid=(NB,),
    in_specs=[_split_spec, _split_spec,
              pl.BlockSpec((H, H), lambda i: (0, 0)),
              pl.BlockSpec((H, H), lambda i: (0, 0))],
    out_specs=[_y_spec, _st_spec],
    out_shape=[jax.ShapeDtypeStruct((N, H), jnp.float32),
               jax.ShapeDtypeStruct((2, H), jnp.float32)],
)

_norm_split = pl.pallas_call(
    _norm_split_body,
    grid=(NB,),
    in_specs=[_y_spec, _st_spec, _vec_spec, _vec_spec],
    out_specs=_split_spec,
    out_shape=jax.ShapeDtypeStruct((2, N, HF), jnp.float32),
)

_norm_full = pl.pallas_call(
    _norm_full_body,
    grid=(NB,),
    in_specs=[_y_spec, _st_spec, _vec_spec, _vec_spec],
    out_specs=_y_spec,
    out_shape=jax.ShapeDtypeStruct((N, H), jnp.float32),
)


def kernel(x, edge_index, Wt, bt, gt, bbn, W1s, W2s, gammas, betas):
    def _pad_idx(row, fill):
        a = row.astype(jnp.int32).reshape(NS, EPT)
        pad = jnp.full((NS, EPAD), fill, jnp.int32)
        return jnp.concatenate([a, pad], axis=1)

    src = _pad_idx(edge_index[0], 0)
    dst = _pad_idx(edge_index[1], N).reshape(NS, NCHUNK, CW)
    zeros = jnp.zeros((RPT, HF), jnp.float32)
    sc_agg = _make_sc_agg()

    y, st = _transform(x, Wt, bt.reshape(1, H))
    hs = _norm_split(y, st, gt.reshape(1, H), bbn.reshape(1, H))
    for i in range(3):
        agg = sc_agg(hs, src, dst, zeros)
        y, st = _layer(hs, agg, W1s[i], W2s[i])
        g = gammas[i].reshape(1, H)
        b = betas[i].reshape(1, H)
        if i < 2:
            hs = _norm_split(y, st, g, b)
        else:
            h = _norm_full(y, st, g, b)
    return h


# speedup vs baseline: 1.2841x; 1.2841x over previous
import functools

import jax
import jax.numpy as jnp
from jax import lax
from jax.experimental import pallas as pl
from jax.experimental.pallas import tpu as pltpu
from jax.experimental.pallas import tpu_sc as plsc

N = 10000
F = 128
H = 256
HF = H // 2
E = 160000
EPS = 1e-5

NS = 16
EPT = E // NS
CW = 125
NCHUNK = EPT // CW
NPAD = 10240
RPT = NPAD // NS

BN = 1000
NB = N // BN


def _sc_agg_body(h_hbm, src_hbm, dst_hbm, zeros_hbm, out_hbm,
                 src_v, dst_v, rows_v, acc, sem):
    c = lax.axis_index("c")
    s = lax.axis_index("s")
    pltpu.sync_copy(src_hbm.at[s], src_v)
    pltpu.sync_copy(dst_hbm.at[s], dst_v)
    pltpu.sync_copy(zeros_hbm, acc.at[pl.ds(s * RPT, RPT)])
    plsc.subcore_barrier()

    def half(h2, o2):
        def chunk(j, carry):
            pltpu.async_copy(h2.at[src_v.at[j]], rows_v, sem).wait()
            pltpu.sync_copy(rows_v, acc.at[dst_v.at[j]], add=True)
            return carry
        lax.fori_loop(0, NCHUNK, chunk, 0)
        plsc.subcore_barrier()
        pltpu.sync_copy(acc.at[pl.ds(s * RPT, RPT)], o2.at[pl.ds(s * RPT, RPT)])

    pl.when(c == 0)(lambda: half(h_hbm.at[0], out_hbm.at[0]))
    pl.when(c == 1)(lambda: half(h_hbm.at[1], out_hbm.at[1]))


def _make_sc_agg():
    mesh = plsc.VectorSubcoreMesh(core_axis_name="c", subcore_axis_name="s")
    return pl.kernel(
        _sc_agg_body,
        out_type=jax.ShapeDtypeStruct((2, NPAD, HF), jnp.float32),
        mesh=mesh,
        scratch_types=[
            pltpu.VMEM((NCHUNK, CW), jnp.int32),
            pltpu.VMEM((NCHUNK, CW), jnp.int32),
            pltpu.VMEM((CW, HF), jnp.float32),
            pltpu.VMEM_SHARED((NPAD, HF), jnp.float32),
            pltpu.SemaphoreType.DMA,
        ],
    )


def _accum_stats(i, y, st_scr):
    st = jnp.concatenate(
        [jnp.sum(y, axis=0, keepdims=True),
         jnp.sum(y * y, axis=0, keepdims=True)], axis=0)

    @pl.when(i == 0)
    def _():
        st_scr[...] = st

    @pl.when(i > 0)
    def _():
        st_scr[...] += st


def _norm(y, st, g, b):
    m = st[0:1, :] * (1.0 / N)
    v = st[1:2, :] * (1.0 / N) - m * m
    inv = lax.rsqrt(v + EPS)
    return (y - m) * (inv * g) + b


def _write_split(o_ref, hn):
    o_ref[0] = hn[:, :HF]
    o_ref[1] = hn[:, HF:]


def _transform_ns_body(x_ref, wt_ref, bt_ref, g_ref, b_ref, o_ref,
                       y_scr, st_scr):
    p = pl.program_id(0)
    i = pl.program_id(1)

    @pl.when(p == 0)
    def _():
        y = jnp.dot(x_ref[...], wt_ref[...],
                    preferred_element_type=jnp.float32) + bt_ref[...]
        y_scr[pl.ds(i * BN, BN), :] = y
        _accum_stats(i, y, st_scr)

    @pl.when(p == 1)
    def _():
        y = y_scr[pl.ds(i * BN, BN), :]
        _write_split(o_ref, _norm(y, st_scr[...], g_ref[...], b_ref[...]))


def _layer_ns_body(h_ref, a_ref, w1_ref, w2_ref, g_ref, b_ref, o_ref,
                   y_scr, st_scr):
    p = pl.program_id(0)
    i = pl.program_id(1)

    @pl.when(p == 0)
    def _():
        s0 = h_ref[0] + a_ref[0]
        s1 = h_ref[1] + a_ref[1]
        u = (jnp.dot(s0, w1_ref[:HF, :], preferred_element_type=jnp.float32)
             + jnp.dot(s1, w1_ref[HF:, :], preferred_element_type=jnp.float32))
        u = jnp.maximum(u, 0.0)
        y = jnp.dot(u, w2_ref[...], preferred_element_type=jnp.float32)
        y = jnp.maximum(y, 0.0)
        y_scr[pl.ds(i * BN, BN), :] = y
        _accum_stats(i, y, st_scr)

    @pl.when(p == 1)
    def _():
        y = y_scr[pl.ds(i * BN, BN), :]
        _write_split(o_ref, _norm(y, st_scr[...], g_ref[...], b_ref[...]))


def _layer_nf_body(h_ref, a_ref, w1_ref, w2_ref, g_ref, b_ref, o_ref,
                   y_scr, st_scr):
    p = pl.program_id(0)
    i = pl.program_id(1)

    @pl.when(p == 0)
    def _():
        s0 = h_ref[0] + a_ref[0]
        s1 = h_ref[1] + a_ref[1]
        u = (jnp.dot(s0, w1_ref[:HF, :], preferred_element_type=jnp.float32)
             + jnp.dot(s1, w1_ref[HF:, :], preferred_element_type=jnp.float32))
        u = jnp.maximum(u, 0.0)
        y = jnp.dot(u, w2_ref[...], preferred_element_type=jnp.float32)
        y = jnp.maximum(y, 0.0)
        y_scr[pl.ds(i * BN, BN), :] = y
        _accum_stats(i, y, st_scr)

    @pl.when(p == 1)
    def _():
        y = y_scr[pl.ds(i * BN, BN), :]
        o_ref[...] = _norm(y, st_scr[...], g_ref[...], b_ref[...])


_vec_spec = pl.BlockSpec((1, H), lambda p, i: (0, 0))
_w_spec = pl.BlockSpec((H, H), lambda p, i: (0, 0))
_split_in_spec = pl.BlockSpec((2, BN, HF), lambda p, i: (0, i * (1 - p), 0))
_split_out_spec = pl.BlockSpec((2, BN, HF), lambda p, i: (0, i * p, 0))
_full_out_spec = pl.BlockSpec((BN, H), lambda p, i: (i * p, 0))
_scratch = [pltpu.VMEM((N, H), jnp.float32), pltpu.VMEM((2, H), jnp.float32)]

_transform_ns = pl.pallas_call(
    _transform_ns_body,
    grid=(2, NB),
    in_specs=[pl.BlockSpec((BN, F), lambda p, i: (i * (1 - p), 0)),
              pl.BlockSpec((F, H), lambda p, i: (0, 0)),
              _vec_spec, _vec_spec, _vec_spec],
    out_specs=_split_out_spec,
    out_shape=jax.ShapeDtypeStruct((2, N, HF), jnp.float32),
    scratch_shapes=_scratch,
)

_layer_ns = pl.pallas_call(
    _layer_ns_body,
    grid=(2, NB),
    in_specs=[_split_in_spec, _split_in_spec, _w_spec, _w_spec,
              _vec_spec, _vec_spec],
    out_specs=_split_out_spec,
    out_shape=jax.ShapeDtypeStruct((2, N, HF), jnp.float32),
    scratch_shapes=_scratch,
)

_layer_nf = pl.pallas_call(
    _layer_nf_body,
    grid=(2, NB),
    in_specs=[_split_in_spec, _split_in_spec, _w_spec, _w_spec,
              _vec_spec, _vec_spec],
    out_specs=_full_out_spec,
    out_shape=jax.ShapeDtypeStruct((N, H), jnp.float32),
    scratch_shapes=_scratch,
)


def kernel(x, edge_index, Wt, bt, gt, bbn, W1s, W2s, gammas, betas):
    src = edge_index[0].astype(jnp.int32).reshape(NS, NCHUNK, CW)
    dst = edge_index[1].astype(jnp.int32).reshape(NS, NCHUNK, CW)
    zeros = jnp.zeros((RPT, HF), jnp.float32)
    sc_agg = _make_sc_agg()

    hs = _transform_ns(x, Wt, bt.reshape(1, H),
                       gt.reshape(1, H), bbn.reshape(1, H))
    for i in range(3):
        agg = sc_agg(hs, src, dst, zeros)
        g = gammas[i].reshape(1, H)
        b = betas[i].reshape(1, H)
        if i < 2:
            hs = _layer_ns(hs, agg, W1s[i], W2s[i], g, b)
        else:
            h = _layer_nf(hs, agg, W1s[i], W2s[i], g, b)
    return h


# speedup vs baseline: 1.9011x; 1.4806x over previous
import functools

import jax
import jax.numpy as jnp
from jax import lax
from jax.experimental import pallas as pl
from jax.experimental.pallas import tpu as pltpu
from jax.experimental.pallas import tpu_sc as plsc

N = 10000
F = 128
H = 256
HF = H // 2
E = 160000
EPS = 1e-5

NS = 16
EPT = E // NS
CW = 125
NCHUNK = EPT // CW
NPAD = 10240
RPT = NPAD // NS

BN = 1000
NB = N // BN


def _sc_agg_body(h_hbm, src_hbm, dst_hbm, zeros_hbm, out_hbm,
                 src_v, dstw, rows_v0, rows_v1, acc,
                 gsem0, gsem1, dsem0, dsem1):
    c = lax.axis_index("c")
    s = lax.axis_index("s")
    pltpu.sync_copy(src_hbm.at[s], src_v)
    pltpu.sync_copy(zeros_hbm, acc.at[pl.ds(s * RPT, RPT)])
    plsc.subcore_barrier()

    def half(h2, o2):
        rows = (rows_v0, rows_v1)
        gsems = (gsem0, gsem1)
        dsems = (dsem0, dsem1)

        def start(j, b):
            pltpu.async_copy(h2.at[src_v.at[j]], rows[b], gsems[b])
            pltpu.async_copy(dst_hbm.at[s, j], dstw.at[b], dsems[b])

        def drain_scatter(j, b):
            pltpu.make_async_copy(h2.at[src_v.at[j]], rows[b], gsems[b]).wait()
            pltpu.make_async_copy(dst_hbm.at[s, j], dstw.at[b], dsems[b]).wait()
            pltpu.sync_copy(rows[b], acc.at[dstw.at[b]], add=True)

        start(0, 0)

        def body2(t, carry):
            j = 2 * t
            start(j + 1, 1)
            drain_scatter(j, 0)
            start(lax.rem(j + 2, NCHUNK), 0)
            drain_scatter(j + 1, 1)
            return carry
        lax.fori_loop(0, NCHUNK // 2, body2, 0)
        pltpu.make_async_copy(h2.at[src_v.at[0]], rows[0], gsems[0]).wait()
        pltpu.make_async_copy(dst_hbm.at[s, 0], dstw.at[0], dsems[0]).wait()
        plsc.subcore_barrier()
        pltpu.sync_copy(acc.at[pl.ds(s * RPT, RPT)], o2.at[pl.ds(s * RPT, RPT)])

    pl.when(c == 0)(lambda: half(h_hbm.at[0], out_hbm.at[0]))
    pl.when(c == 1)(lambda: half(h_hbm.at[1], out_hbm.at[1]))


def _make_sc_agg():
    mesh = plsc.VectorSubcoreMesh(core_axis_name="c", subcore_axis_name="s")
    return pl.kernel(
        _sc_agg_body,
        out_type=jax.ShapeDtypeStruct((2, NPAD, HF), jnp.float32),
        mesh=mesh,
        scratch_types=[
            pltpu.VMEM((NCHUNK, CW), jnp.int32),
            pltpu.VMEM((2, CW), jnp.int32),
            pltpu.VMEM((CW, HF), jnp.float32),
            pltpu.VMEM((CW, HF), jnp.float32),
            pltpu.VMEM_SHARED((NPAD, HF), jnp.float32),
            pltpu.SemaphoreType.DMA,
            pltpu.SemaphoreType.DMA,
            pltpu.SemaphoreType.DMA,
            pltpu.SemaphoreType.DMA,
        ],
    )


def _accum_stats(i, y, st_scr):
    st = jnp.concatenate(
        [jnp.sum(y, axis=0, keepdims=True),
         jnp.sum(y * y, axis=0, keepdims=True)], axis=0)

    @pl.when(i == 0)
    def _():
        st_scr[...] = st

    @pl.when(i > 0)
    def _():
        st_scr[...] += st


def _norm(y, st, g, b):
    m = st[0:1, :] * (1.0 / N)
    v = st[1:2, :] * (1.0 / N) - m * m
    inv = lax.rsqrt(v + EPS)
    return (y - m) * (inv * g) + b


def _write_split(o_ref, hn):
    o_ref[0] = hn[:, :HF]
    o_ref[1] = hn[:, HF:]


def _transform_ns_body(x_ref, wt_ref, bt_ref, g_ref, b_ref, o_ref,
                       y_scr, st_scr):
    p = pl.program_id(0)
    i = pl.program_id(1)

    @pl.when(p == 0)
    def _():
        y = jnp.dot(x_ref[...], wt_ref[...],
                    preferred_element_type=jnp.float32) + bt_ref[...]
        y_scr[pl.ds(i * BN, BN), :] = y
        _accum_stats(i, y, st_scr)

    @pl.when(p == 1)
    def _():
        y = y_scr[pl.ds(i * BN, BN), :]
        _write_split(o_ref, _norm(y, st_scr[...], g_ref[...], b_ref[...]))


def _layer_ns_body(h_ref, a_ref, w1_ref, w2_ref, g_ref, b_ref, o_ref,
                   y_scr, st_scr):
    p = pl.program_id(0)
    i = pl.program_id(1)

    @pl.when(p == 0)
    def _():
        s0 = h_ref[0] + a_ref[0]
        s1 = h_ref[1] + a_ref[1]
        u = (jnp.dot(s0, w1_ref[:HF, :], preferred_element_type=jnp.float32)
             + jnp.dot(s1, w1_ref[HF:, :], preferred_element_type=jnp.float32))
        u = jnp.maximum(u, 0.0)
        y = jnp.dot(u, w2_ref[...], preferred_element_type=jnp.float32)
        y = jnp.maximum(y, 0.0)
        y_scr[pl.ds(i * BN, BN), :] = y
        _accum_stats(i, y, st_scr)

    @pl.when(p == 1)
    def _():
        y = y_scr[pl.ds(i * BN, BN), :]
        _write_split(o_ref, _norm(y, st_scr[...], g_ref[...], b_ref[...]))


def _layer_nf_body(h_ref, a_ref, w1_ref, w2_ref, g_ref, b_ref, o_ref,
                   y_scr, st_scr):
    p = pl.program_id(0)
    i = pl.program_id(1)

    @pl.when(p == 0)
    def _():
        s0 = h_ref[0] + a_ref[0]
        s1 = h_ref[1] + a_ref[1]
        u = (jnp.dot(s0, w1_ref[:HF, :], preferred_element_type=jnp.float32)
             + jnp.dot(s1, w1_ref[HF:, :], preferred_element_type=jnp.float32))
        u = jnp.maximum(u, 0.0)
        y = jnp.dot(u, w2_ref[...], preferred_element_type=jnp.float32)
        y = jnp.maximum(y, 0.0)
        y_scr[pl.ds(i * BN, BN), :] = y
        _accum_stats(i, y, st_scr)

    @pl.when(p == 1)
    def _():
        y = y_scr[pl.ds(i * BN, BN), :]
        o_ref[...] = _norm(y, st_scr[...], g_ref[...], b_ref[...])


_vec_spec = pl.BlockSpec((1, H), lambda p, i: (0, 0))
_w_spec = pl.BlockSpec((H, H), lambda p, i: (0, 0))
_split_in_spec = pl.BlockSpec((2, BN, HF), lambda p, i: (0, i * (1 - p), 0))
_split_out_spec = pl.BlockSpec((2, BN, HF), lambda p, i: (0, i * p, 0))
_full_out_spec = pl.BlockSpec((BN, H), lambda p, i: (i * p, 0))
_scratch = [pltpu.VMEM((N, H), jnp.float32), pltpu.VMEM((2, H), jnp.float32)]

_transform_ns = pl.pallas_call(
    _transform_ns_body,
    grid=(2, NB),
    in_specs=[pl.BlockSpec((BN, F), lambda p, i: (i * (1 - p), 0)),
              pl.BlockSpec((F, H), lambda p, i: (0, 0)),
              _vec_spec, _vec_spec, _vec_spec],
    out_specs=_split_out_spec,
    out_shape=jax.ShapeDtypeStruct((2, N, HF), jnp.float32),
    scratch_shapes=_scratch,
)

_layer_ns = pl.pallas_call(
    _layer_ns_body,
    grid=(2, NB),
    in_specs=[_split_in_spec, _split_in_spec, _w_spec, _w_spec,
              _vec_spec, _vec_spec],
    out_specs=_split_out_spec,
    out_shape=jax.ShapeDtypeStruct((2, N, HF), jnp.float32),
    scratch_shapes=_scratch,
)

_layer_nf = pl.pallas_call(
    _layer_nf_body,
    grid=(2, NB),
    in_specs=[_split_in_spec, _split_in_spec, _w_spec, _w_spec,
              _vec_spec, _vec_spec],
    out_specs=_full_out_spec,
    out_shape=jax.ShapeDtypeStruct((N, H), jnp.float32),
    scratch_shapes=_scratch,
)


def kernel(x, edge_index, Wt, bt, gt, bbn, W1s, W2s, gammas, betas):
    src = edge_index[0].astype(jnp.int32).reshape(NS, NCHUNK, CW)
    dst = edge_index[1].astype(jnp.int32).reshape(NS, NCHUNK, CW)
    zeros = jnp.zeros((RPT, HF), jnp.float32)
    sc_agg = _make_sc_agg()

    hs = _transform_ns(x, Wt, bt.reshape(1, H),
                       gt.reshape(1, H), bbn.reshape(1, H))
    for i in range(3):
        agg = sc_agg(hs, src, dst, zeros)
        g = gammas[i].reshape(1, H)
        b = betas[i].reshape(1, H)
        if i < 2:
            hs = _layer_ns(hs, agg, W1s[i], W2s[i], g, b)
        else:
            h = _layer_nf(hs, agg, W1s[i], W2s[i], g, b)
    return h


# speedup vs baseline: 1.9073x; 1.0033x over previous
import jax
import jax.numpy as jnp
from jax import lax
from jax.experimental import pallas as pl
from jax.experimental.pallas import tpu as pltpu
from jax.experimental.pallas import tpu_sc as plsc

N = 10000
F = 128
H = 256
HF = H // 2
E = 160000
EPS = 1e-5

NS = 16
EPT = E // NS
CW = 125
NCHUNK = EPT // CW
NPAD = 10240
RPT = NPAD // NS

BN = 1000
NB = N // BN


def _sc_agg_body(h_hbm, src_hbm, dst_hbm, zeros_hbm, out_hbm,
                 src_v, dstw, rows_v0, rows_v1, acc,
                 gsem0, gsem1, dsem0, dsem1):
    c = lax.axis_index("c")
    s = lax.axis_index("s")
    pltpu.sync_copy(src_hbm.at[s], src_v)
    pltpu.sync_copy(zeros_hbm, acc.at[pl.ds(s * RPT, RPT)])
    plsc.subcore_barrier()

    def half(h2, o2):
        rows = (rows_v0, rows_v1)
        gsems = (gsem0, gsem1)
        dsems = (dsem0, dsem1)

        def start(j, b):
            pltpu.async_copy(h2.at[src_v.at[j]], rows[b], gsems[b])
            pltpu.async_copy(dst_hbm.at[s, j], dstw.at[b], dsems[b])

        def drain_scatter(j, b):
            pltpu.make_async_copy(h2.at[src_v.at[j]], rows[b], gsems[b]).wait()
            pltpu.make_async_copy(dst_hbm.at[s, j], dstw.at[b], dsems[b]).wait()
            pltpu.sync_copy(rows[b], acc.at[dstw.at[b]], add=True)

        start(0, 0)

        def body2(t, carry):
            j = 2 * t
            start(j + 1, 1)
            drain_scatter(j, 0)
            start(lax.rem(j + 2, NCHUNK), 0)
            drain_scatter(j + 1, 1)
            return carry
        lax.fori_loop(0, NCHUNK // 2, body2, 0)
        pltpu.make_async_copy(h2.at[src_v.at[0]], rows[0], gsems[0]).wait()
        pltpu.make_async_copy(dst_hbm.at[s, 0], dstw.at[0], dsems[0]).wait()
        plsc.subcore_barrier()
        pltpu.sync_copy(acc.at[pl.ds(s * RPT, RPT)], o2.at[pl.ds(s * RPT, RPT)])

    pl.when(c == 0)(lambda: half(h_hbm.at[0], out_hbm.at[0]))
    pl.when(c == 1)(lambda: half(h_hbm.at[1], out_hbm.at[1]))


def _make_sc_agg():
    mesh = plsc.VectorSubcoreMesh(core_axis_name="c", subcore_axis_name="s")
    return pl.kernel(
        _sc_agg_body,
        out_type=jax.ShapeDtypeStruct((2, NPAD, HF), jnp.float32),
        mesh=mesh,
        scratch_types=[
            pltpu.VMEM((NCHUNK, CW), jnp.int32),
            pltpu.VMEM((2, CW), jnp.int32),
            pltpu.VMEM((CW, HF), jnp.float32),
            pltpu.VMEM((CW, HF), jnp.float32),
            pltpu.VMEM_SHARED((NPAD, HF), jnp.float32),
            pltpu.SemaphoreType.DMA,
            pltpu.SemaphoreType.DMA,
            pltpu.SemaphoreType.DMA,
            pltpu.SemaphoreType.DMA,
        ],
    )


def _accum_stats(i, y, st_scr):
    st = jnp.concatenate(
        [jnp.sum(y, axis=0, keepdims=True),
         jnp.sum(y * y, axis=0, keepdims=True)], axis=0)

    @pl.when(i == 0)
    def _():
        st_scr[...] = st

    @pl.when(i > 0)
    def _():
        st_scr[...] += st


def _norm(y, st, g, b):
    m = st[0:1, :] * (1.0 / N)
    v = st[1:2, :] * (1.0 / N) - m * m
    inv = lax.rsqrt(v + EPS)
    return (y - m) * (inv * g) + b


def _write_split(o_ref, hn):
    o_ref[0] = hn[:, :HF]
    o_ref[1] = hn[:, HF:]


def _transform_ns_body(x_ref, wt_ref, bt_ref, g_ref, b_ref, o_ref,
                       y_scr, st_scr):
    p = pl.program_id(0)
    i = pl.program_id(1)

    @pl.when(p == 0)
    def _():
        y = jnp.dot(x_ref[...], wt_ref[...],
                    preferred_element_type=jnp.float32) + bt_ref[...]
        y_scr[pl.ds(i * BN, BN), :] = y
        _accum_stats(i, y, st_scr)

    @pl.when(p == 1)
    def _():
        y = y_scr[pl.ds(i * BN, BN), :]
        _write_split(o_ref, _norm(y, st_scr[...], g_ref[...], b_ref[...]))


def _layer_ns_body(h_ref, a_ref, w1_ref, w2_ref, g_ref, b_ref, o_ref,
                   y_scr, st_scr):
    p = pl.program_id(0)
    i = pl.program_id(1)

    @pl.when(p == 0)
    def _():
        s0 = h_ref[0] + a_ref[0]
        s1 = h_ref[1] + a_ref[1]
        u = (jnp.dot(s0, w1_ref[:HF, :], preferred_element_type=jnp.float32)
             + jnp.dot(s1, w1_ref[HF:, :], preferred_element_type=jnp.float32))
        u = jnp.maximum(u, 0.0)
        y = jnp.dot(u, w2_ref[...], preferred_element_type=jnp.float32)
        y = jnp.maximum(y, 0.0)
        y_scr[pl.ds(i * BN, BN), :] = y
        _accum_stats(i, y, st_scr)

    @pl.when(p == 1)
    def _():
        y = y_scr[pl.ds(i * BN, BN), :]
        _write_split(o_ref, _norm(y, st_scr[...], g_ref[...], b_ref[...]))


def _layer_nf_body(h_ref, a_ref, w1_ref, w2_ref, g_ref, b_ref, o_ref,
                   y_scr, st_scr):
    p = pl.program_id(0)
    i = pl.program_id(1)

    @pl.when(p == 0)
    def _():
        s0 = h_ref[0] + a_ref[0]
        s1 = h_ref[1] + a_ref[1]
        u = (jnp.dot(s0, w1_ref[:HF, :], preferred_element_type=jnp.float32)
             + jnp.dot(s1, w1_ref[HF:, :], preferred_element_type=jnp.float32))
        u = jnp.maximum(u, 0.0)
        y = jnp.dot(u, w2_ref[...], preferred_element_type=jnp.float32)
        y = jnp.maximum(y, 0.0)
        y_scr[pl.ds(i * BN, BN), :] = y
        _accum_stats(i, y, st_scr)

    @pl.when(p == 1)
    def _():
        y = y_scr[pl.ds(i * BN, BN), :]
        o_ref[...] = _norm(y, st_scr[...], g_ref[...], b_ref[...])


_vec_spec = pl.BlockSpec((1, H), lambda p, i: (0, 0))
_w_spec = pl.BlockSpec((H, H), lambda p, i: (0, 0))
_split_in_spec = pl.BlockSpec((2, BN, HF), lambda p, i: (0, i * (1 - p), 0))
_split_out_spec = pl.BlockSpec((2, BN, HF), lambda p, i: (0, i * p, 0))
_full_out_spec = pl.BlockSpec((BN, H), lambda p, i: (i * p, 0))
_scratch = [pltpu.VMEM((N, H), jnp.float32), pltpu.VMEM((2, H), jnp.float32)]

_transform_ns = pl.pallas_call(
    _transform_ns_body,
    grid=(2, NB),
    in_specs=[pl.BlockSpec((BN, F), lambda p, i: (i * (1 - p), 0)),
              pl.BlockSpec((F, H), lambda p, i: (0, 0)),
              _vec_spec, _vec_spec, _vec_spec],
    out_specs=_split_out_spec,
    out_shape=jax.ShapeDtypeStruct((2, N, HF), jnp.float32),
    scratch_shapes=_scratch,
)

_layer_ns = pl.pallas_call(
    _layer_ns_body,
    grid=(2, NB),
    in_specs=[_split_in_spec, _split_in_spec, _w_spec, _w_spec,
              _vec_spec, _vec_spec],
    out_specs=_split_out_spec,
    out_shape=jax.ShapeDtypeStruct((2, N, HF), jnp.float32),
    scratch_shapes=_scratch,
)

_layer_nf = pl.pallas_call(
    _layer_nf_body,
    grid=(2, NB),
    in_specs=[_split_in_spec, _split_in_spec, _w_spec, _w_spec,
              _vec_spec, _vec_spec],
    out_specs=_full_out_spec,
    out_shape=jax.ShapeDtypeStruct((N, H), jnp.float32),
    scratch_shapes=_scratch,
)


def kernel(x, edge_index, Wt, bt, gt, bbn, W1s, W2s, gammas, betas):
    src = edge_index[0].astype(jnp.int32).reshape(NS, NCHUNK, CW)
    dst = edge_index[1].astype(jnp.int32).reshape(NS, NCHUNK, CW)
    zeros = jnp.zeros((RPT, HF), jnp.float32)
    sc_agg = _make_sc_agg()

    hs = _transform_ns(x, Wt, bt.reshape(1, H),
                       gt.reshape(1, H), bbn.reshape(1, H))
    for i in range(3):
        agg = sc_agg(hs, src, dst, zeros)
        g = gammas[i].reshape(1, H)
        b = betas[i].reshape(1, H)
        if i < 2:
            hs = _layer_ns(hs, agg, W1s[i], W2s[i], g, b)
        else:
            h = _layer_nf(hs, agg, W1s[i], W2s[i], g, b)
    return h
